# Initial kernel scaffold; baseline (speedup 1.0000x reference)
import jax
import jax.numpy as jnp
from jax.experimental import pallas as pl


def kernel(x, edge_index, W1, b1, W2, b2, Wc, bc):
    raise NotImplementedError("write your pallas kernel here")


# speedup vs baseline: 6.1238x; 6.1238x over previous
import functools

import jax
import jax.numpy as jnp
from jax import lax
from jax.experimental import pallas as pl
from jax.experimental.pallas import tpu as pltpu
from jax.experimental.pallas import tpu_sc as plsc

N_NODES_C = 10000
N_PAD = 10240
N_EDGES_C = 320000
D1 = 128
DH = 64
W1TAB = 80

NC = 2
NS = 16
NW = NC * NS
EDGES_PER_TILE = N_EDGES_C // NW
CHUNK = 80
N_CHUNKS = EDGES_PER_TILE // CHUNK
ROWS_PER_TILE = N_PAD // NS


def _tc1_body(x_ref, w1_ref, out_ref):
    mm = jnp.dot(x_ref[...], w1_ref[...], preferred_element_type=jnp.float32)
    n = mm.shape[0]
    col = lax.broadcasted_iota(jnp.int32, (n, W1TAB - DH), 1)
    extra = jnp.where(col == 0, 1.0, 0.0).astype(jnp.float32)
    out_ref[...] = jnp.concatenate([mm, extra], axis=1)


def _tc2_body(agg_ref, w2_ref, b1_ref, t2_ref):
    agg = (agg_ref[0] + agg_ref[1])[:N_NODES_C]
    deg = jnp.maximum(agg[:, DH:DH + 1], 1.0)
    h1 = jax.nn.relu(agg[:, :DH] / deg + b1_ref[...])
    t2_ref[...] = jnp.dot(h1, w2_ref[...], preferred_element_type=jnp.float32)


def _tc3_body(agg2_ref, agg1_ref, b2_ref, wc_ref, bc_ref, out_ref):
    agg = (agg2_ref[0] + agg2_ref[1])[:N_NODES_C]
    dcol = (agg1_ref[0] + agg1_ref[1])[:N_NODES_C]
    deg = jnp.maximum(dcol[:, DH:DH + 1], 1.0)
    h2 = jax.nn.relu(agg / deg + b2_ref[...])
    hg = jnp.sum(h2, axis=0, keepdims=True) / N_NODES_C
    out_ref[...] = jnp.sum(hg * wc_ref[...], axis=1, keepdims=True) + bc_ref[...]


def _sc_agg_body(width, table, src, dst, zrows, out, acc, sidx, didx, rows, sem):
    c = lax.axis_index("c")
    s = lax.axis_index("s")
    wid = c * NS + s
    ebase = wid * EDGES_PER_TILE
    rbase = s * ROWS_PER_TILE

    pltpu.sync_copy(zrows, acc.at[pl.ds(rbase, ROWS_PER_TILE)])
    plsc.subcore_barrier()

    @pl.loop(0, N_CHUNKS)
    def _chunk(i):
        base = ebase + i * CHUNK
        pltpu.sync_copy(src.at[pl.ds(base, CHUNK)], sidx)
        pltpu.sync_copy(dst.at[pl.ds(base, CHUNK)], didx)
        pltpu.async_copy(table.at[sidx], rows, sem).wait()
        pltpu.sync_copy(rows, acc.at[didx], add=True)

    plsc.subcore_barrier()
    pltpu.sync_copy(acc.at[pl.ds(rbase, ROWS_PER_TILE)],
                    out.at[c, pl.ds(rbase, ROWS_PER_TILE)])


def _make_sc_agg(width):
    mesh = plsc.VectorSubcoreMesh(core_axis_name="c", subcore_axis_name="s")
    return pl.kernel(
        functools.partial(_sc_agg_body, width),
        out_type=jax.ShapeDtypeStruct((NC, N_PAD, width), jnp.float32),
        mesh=mesh,
        scratch_types=[
            pltpu.VMEM_SHARED((N_PAD, width), jnp.float32),
            pltpu.VMEM((CHUNK,), jnp.int32),
            pltpu.VMEM((CHUNK,), jnp.int32),
            pltpu.VMEM((CHUNK, width), jnp.float32),
            pltpu.SemaphoreType.DMA,
        ],
        compiler_params=pltpu.CompilerParams(use_tc_tiling_on_sc=False),
    )


_sc_agg_80 = _make_sc_agg(W1TAB)
_sc_agg_64 = _make_sc_agg(DH)


def kernel(x, edge_index, W1, b1, W2, b2, Wc, bc):
    src = edge_index[0].astype(jnp.int32)
    dst = edge_index[1].astype(jnp.int32)
    zrows = jnp.zeros((ROWS_PER_TILE, W1TAB), jnp.float32)
    zrows64 = jnp.zeros((ROWS_PER_TILE, DH), jnp.float32)

    t1 = pl.pallas_call(
        _tc1_body,
        out_shape=jax.ShapeDtypeStruct((N_NODES_C, W1TAB), jnp.float32),
    )(x, W1)

    agg1 = _sc_agg_80(t1, src, dst, zrows)

    t2 = pl.pallas_call(
        _tc2_body,
        out_shape=jax.ShapeDtypeStruct((N_NODES_C, DH), jnp.float32),
    )(agg1, W2, b1.reshape(1, DH))

    agg2 = _sc_agg_64(t2, src, dst, zrows64)

    out = pl.pallas_call(
        _tc3_body,
        out_shape=jax.ShapeDtypeStruct((1, 1), jnp.float32),
    )(agg2, agg1, b2.reshape(1, DH), Wc.reshape(1, DH), bc.reshape(1, 1))
    return out


# speedup vs baseline: 12.7137x; 2.0761x over previous
import functools

import jax
import jax.numpy as jnp
from jax import lax
from jax.experimental import pallas as pl
from jax.experimental.pallas import tpu as pltpu
from jax.experimental.pallas import tpu_sc as plsc

N_NODES_C = 10000
N_PAD = 10240
N_EDGES_C = 320000
D1 = 128
DH = 64
W1TAB = 80

NC = 2
NS = 16
NW = NC * NS
EDGES_PER_TILE = N_EDGES_C // NW
CHUNK = 80
N_CHUNKS = EDGES_PER_TILE // CHUNK
ROWS_PER_TILE = N_PAD // NS


def _tc1_body(x_ref, w1_ref, out_ref):
    mm = jnp.dot(x_ref[...], w1_ref[...], preferred_element_type=jnp.float32)
    n = mm.shape[0]
    col = lax.broadcasted_iota(jnp.int32, (n, W1TAB - DH), 1)
    extra = jnp.where(col == 0, 1.0, 0.0).astype(jnp.float32)
    out_ref[...] = jnp.concatenate([mm, extra], axis=1)


def _tc2_body(agg_ref, w2_ref, b1_ref, t2_ref):
    agg = (agg_ref[0] + agg_ref[1])[:N_NODES_C]
    deg = jnp.maximum(agg[:, DH:DH + 1], 1.0)
    h1 = jax.nn.relu(agg[:, :DH] / deg + b1_ref[...])
    t2_ref[...] = jnp.dot(h1, w2_ref[...], preferred_element_type=jnp.float32)


def _tc3_body(agg2_ref, agg1_ref, b2_ref, wc_ref, bc_ref, out_ref):
    agg = (agg2_ref[0] + agg2_ref[1])[:N_NODES_C]
    dcol = (agg1_ref[0] + agg1_ref[1])[:N_NODES_C]
    deg = jnp.maximum(dcol[:, DH:DH + 1], 1.0)
    h2 = jax.nn.relu(agg / deg + b2_ref[...])
    hg = jnp.sum(h2, axis=0, keepdims=True) / N_NODES_C
    out_ref[...] = jnp.sum(hg * wc_ref[...], axis=1, keepdims=True) + bc_ref[...]


NBUF = 5


def _sc_agg_body(width, table, edges, zrows, out, acc, ibufs, rbufs, isems, gsems):
    c = lax.axis_index("c")
    s = lax.axis_index("s")
    wid = c * NS + s
    ebase = wid * EDGES_PER_TILE
    rbase = s * ROWS_PER_TILE

    pltpu.sync_copy(zrows, acc.at[pl.ds(rbase, ROWS_PER_TILE)])
    plsc.subcore_barrier()

    def start_chunk(g, b):
        base = ebase + g * CHUNK
        pltpu.async_copy(edges.at[:, pl.ds(base, CHUNK)], ibufs[b], isems[b]).wait()
        pltpu.async_copy(table.at[ibufs[b].at[0]], rbufs[b], gsems[b])

    def finish_chunk(b):
        pltpu.make_async_copy(table.at[ibufs[b].at[0]], rbufs[b], gsems[b]).wait()
        pltpu.sync_copy(rbufs[b], acc.at[ibufs[b].at[1]], add=True)

    for b in range(NBUF):
        start_chunk(b, b)

    @pl.loop(0, N_CHUNKS - NBUF, step=NBUF)
    def _chunks(i):
        for b in range(NBUF):
            finish_chunk(b)
            start_chunk(i + b + NBUF, b)

    for b in range(NBUF):
        finish_chunk(b)

    plsc.subcore_barrier()
    pltpu.sync_copy(acc.at[pl.ds(rbase, ROWS_PER_TILE)],
                    out.at[c, pl.ds(rbase, ROWS_PER_TILE)])


def _make_sc_agg(width):
    mesh = plsc.VectorSubcoreMesh(core_axis_name="c", subcore_axis_name="s")
    return pl.kernel(
        functools.partial(_sc_agg_body, width),
        out_type=jax.ShapeDtypeStruct((NC, N_PAD, width), jnp.float32),
        mesh=mesh,
        scratch_types=[
            pltpu.VMEM_SHARED((N_PAD, width), jnp.float32),
            [pltpu.VMEM((2, CHUNK), jnp.int32) for _ in range(NBUF)],
            [pltpu.VMEM((CHUNK, width), jnp.float32) for _ in range(NBUF)],
            [pltpu.SemaphoreType.DMA for _ in range(NBUF)],
            [pltpu.SemaphoreType.DMA for _ in range(NBUF)],
        ],
        compiler_params=pltpu.CompilerParams(use_tc_tiling_on_sc=False),
    )


_sc_agg_80 = _make_sc_agg(W1TAB)
_sc_agg_64 = _make_sc_agg(DH)


def kernel(x, edge_index, W1, b1, W2, b2, Wc, bc):
    edges = edge_index.astype(jnp.int32)
    zrows = jnp.zeros((ROWS_PER_TILE, W1TAB), jnp.float32)
    zrows64 = jnp.zeros((ROWS_PER_TILE, DH), jnp.float32)

    t1 = pl.pallas_call(
        _tc1_body,
        out_shape=jax.ShapeDtypeStruct((N_NODES_C, W1TAB), jnp.float32),
    )(x, W1)

    agg1 = _sc_agg_80(t1, edges, zrows)

    t2 = pl.pallas_call(
        _tc2_body,
        out_shape=jax.ShapeDtypeStruct((N_NODES_C, DH), jnp.float32),
    )(agg1, W2, b1.reshape(1, DH))

    agg2 = _sc_agg_64(t2, edges, zrows64)

    out = pl.pallas_call(
        _tc3_body,
        out_shape=jax.ShapeDtypeStruct((1, 1), jnp.float32),
    )(agg2, agg1, b2.reshape(1, DH), Wc.reshape(1, DH), bc.reshape(1, 1))
    return out


# speedup vs baseline: 13.7941x; 1.0850x over previous
import functools

import jax
import jax.numpy as jnp
from jax import lax
from jax.experimental import pallas as pl
from jax.experimental.pallas import tpu as pltpu
from jax.experimental.pallas import tpu_sc as plsc

N_NODES_C = 10000
N_PAD = 10240
N_EDGES_C = 320000
D1 = 128
DH = 64
W1TAB = 80

NC = 2
NS = 16
NW = NC * NS
EDGES_PER_TILE = N_EDGES_C // NW
CHUNK = 80
N_CHUNKS = EDGES_PER_TILE // CHUNK
ROWS_PER_TILE = N_PAD // NS


def _tc1_body(x_ref, w1_ref, out_ref):
    mm = jnp.dot(x_ref[...], w1_ref[...], preferred_element_type=jnp.float32)
    n = mm.shape[0]
    col = lax.broadcasted_iota(jnp.int32, (n, W1TAB - DH), 1)
    extra = jnp.where(col == 0, 1.0, 0.0).astype(jnp.float32)
    out_ref[...] = jnp.concatenate([mm, extra], axis=1)


def _tc2_body(agg_ref, w2_ref, b1_ref, t2_ref):
    agg = (agg_ref[0] + agg_ref[1])[:N_NODES_C]
    deg = jnp.maximum(agg[:, DH:DH + 1], 1.0)
    h1 = jax.nn.relu(agg[:, :DH] / deg + b1_ref[...])
    t2_ref[...] = jnp.dot(h1, w2_ref[...], preferred_element_type=jnp.float32)


def _tc3_body(agg2_ref, agg1_ref, b2_ref, wc_ref, bc_ref, out_ref):
    agg = (agg2_ref[0] + agg2_ref[1])[:N_NODES_C]
    dcol = (agg1_ref[0] + agg1_ref[1])[:N_NODES_C]
    deg = jnp.maximum(dcol[:, DH:DH + 1], 1.0)
    h2 = jax.nn.relu(agg / deg + b2_ref[...])
    hg = jnp.sum(h2, axis=0, keepdims=True) / N_NODES_C
    out_ref[...] = jnp.sum(hg * wc_ref[...], axis=1, keepdims=True) + bc_ref[...]


NBUF = 5


def _sc_agg_body(width, table, edges, zrows, out, acc, ibufs, rbufs, isems, gsems,
                 ssems):
    c = lax.axis_index("c")
    s = lax.axis_index("s")
    wid = c * NS + s
    ebase = wid * EDGES_PER_TILE
    rbase = s * ROWS_PER_TILE

    pltpu.sync_copy(zrows, acc.at[pl.ds(rbase, ROWS_PER_TILE)])
    plsc.subcore_barrier()

    def start_chunk(g, b):
        base = ebase + g * CHUNK
        pltpu.async_copy(edges.at[:, pl.ds(base, CHUNK)], ibufs[b], isems[b]).wait()
        pltpu.async_copy(table.at[ibufs[b].at[0]], rbufs[b], gsems[b])

    def start_scatter(b):
        pltpu.make_async_copy(table.at[ibufs[b].at[0]], rbufs[b], gsems[b]).wait()
        pltpu.async_copy(rbufs[b], acc.at[ibufs[b].at[1]], ssems[b], add=True)

    def wait_scatter(b):
        pltpu.make_async_copy(rbufs[b], acc.at[ibufs[b].at[1]], ssems[b]).wait()

    for b in range(NBUF):
        start_chunk(b, b)

    @pl.loop(0, N_CHUNKS - NBUF, step=NBUF)
    def _chunks(i):
        for b in range(NBUF):
            start_scatter(b)
        for b in range(NBUF):
            wait_scatter(b)
            start_chunk(i + b + NBUF, b)

    for b in range(NBUF):
        start_scatter(b)
    for b in range(NBUF):
        wait_scatter(b)

    plsc.subcore_barrier()
    pltpu.sync_copy(acc.at[pl.ds(rbase, ROWS_PER_TILE)],
                    out.at[c, pl.ds(rbase, ROWS_PER_TILE)])


def _make_sc_agg(width):
    mesh = plsc.VectorSubcoreMesh(core_axis_name="c", subcore_axis_name="s")
    return pl.kernel(
        functools.partial(_sc_agg_body, width),
        out_type=jax.ShapeDtypeStruct((NC, N_PAD, width), jnp.float32),
        mesh=mesh,
        scratch_types=[
            pltpu.VMEM_SHARED((N_PAD, width), jnp.float32),
            [pltpu.VMEM((2, CHUNK), jnp.int32) for _ in range(NBUF)],
            [pltpu.VMEM((CHUNK, width), jnp.float32) for _ in range(NBUF)],
            [pltpu.SemaphoreType.DMA for _ in range(NBUF)],
            [pltpu.SemaphoreType.DMA for _ in range(NBUF)],
            [pltpu.SemaphoreType.DMA for _ in range(NBUF)],
        ],
        compiler_params=pltpu.CompilerParams(use_tc_tiling_on_sc=False),
    )


_sc_agg_80 = _make_sc_agg(W1TAB)
_sc_agg_64 = _make_sc_agg(DH)


def kernel(x, edge_index, W1, b1, W2, b2, Wc, bc):
    edges = edge_index.astype(jnp.int32)
    zrows = jnp.zeros((ROWS_PER_TILE, W1TAB), jnp.float32)
    zrows64 = jnp.zeros((ROWS_PER_TILE, DH), jnp.float32)

    t1 = pl.pallas_call(
        _tc1_body,
        out_shape=jax.ShapeDtypeStruct((N_NODES_C, W1TAB), jnp.float32),
    )(x, W1)

    agg1 = _sc_agg_80(t1, edges, zrows)

    t2 = pl.pallas_call(
        _tc2_body,
        out_shape=jax.ShapeDtypeStruct((N_NODES_C, DH), jnp.float32),
    )(agg1, W2, b1.reshape(1, DH))

    agg2 = _sc_agg_64(t2, edges, zrows64)

    out = pl.pallas_call(
        _tc3_body,
        out_shape=jax.ShapeDtypeStruct((1, 1), jnp.float32),
    )(agg2, agg1, b2.reshape(1, DH), Wc.reshape(1, DH), bc.reshape(1, 1))
    return out


# speedup vs baseline: 17.2695x; 1.2520x over previous
import functools

import jax
import jax.numpy as jnp
from jax import lax
from jax.experimental import pallas as pl
from jax.experimental.pallas import tpu as pltpu
from jax.experimental.pallas import tpu_sc as plsc

N_NODES_C = 10000
N_PAD = 10240
N_EDGES_C = 320000
D1 = 128
DH = 64
W1TAB = 80

NC = 2
NS = 16
NW = NC * NS
EDGES_PER_TILE = N_EDGES_C // NW
CHUNK = 125
N_CHUNKS = EDGES_PER_TILE // CHUNK
ROWS_PER_TILE = N_PAD // NS


def _tc1_body(x_ref, w1_ref, out_ref):
    mm = jnp.dot(x_ref[...], w1_ref[...], preferred_element_type=jnp.float32)
    n = mm.shape[0]
    col = lax.broadcasted_iota(jnp.int32, (n, W1TAB - DH), 1)
    extra = jnp.where(col == 0, 1.0, 0.0).astype(jnp.float32)
    out_ref[...] = jnp.concatenate([mm, extra], axis=1)


def _tc2_body(agg_ref, w2_ref, b1_ref, t2_ref):
    agg = (agg_ref[0] + agg_ref[1])[:N_NODES_C]
    deg = jnp.maximum(agg[:, DH:DH + 1], 1.0)
    h1 = jax.nn.relu(agg[:, :DH] / deg + b1_ref[...])
    t2_ref[...] = jnp.dot(h1, w2_ref[...], preferred_element_type=jnp.float32)


def _tc3_body(agg2_ref, agg1_ref, b2_ref, wc_ref, bc_ref, out_ref):
    agg = (agg2_ref[0] + agg2_ref[1])[:N_NODES_C]
    dcol = (agg1_ref[0] + agg1_ref[1])[:N_NODES_C]
    deg = jnp.maximum(dcol[:, DH:DH + 1], 1.0)
    h2 = jax.nn.relu(agg / deg + b2_ref[...])
    hg = jnp.sum(h2, axis=0, keepdims=True) / N_NODES_C
    out_ref[...] = jnp.sum(hg * wc_ref[...], axis=1, keepdims=True) + bc_ref[...]


NBUF = 5


def _sc_agg_body(width, table, edges, zrows, out, acc, ibig, rbufs, gsems, ssems):
    c = lax.axis_index("c")
    s = lax.axis_index("s")
    wid = c * NS + s
    rbase = s * ROWS_PER_TILE

    pltpu.sync_copy(edges.at[0, wid], ibig.at[0])
    pltpu.sync_copy(edges.at[1, wid], ibig.at[1])
    pltpu.sync_copy(zrows, acc.at[pl.ds(rbase, ROWS_PER_TILE)])
    plsc.subcore_barrier()

    def start_gather(g, b):
        pltpu.async_copy(table.at[ibig.at[0, g]], rbufs[b], gsems[b])

    def start_scatter(g, b):
        pltpu.make_async_copy(table.at[ibig.at[0, g]], rbufs[b], gsems[b]).wait()
        pltpu.async_copy(rbufs[b], acc.at[ibig.at[1, g]], ssems[b], add=True)

    def wait_scatter(g, b):
        pltpu.make_async_copy(rbufs[b], acc.at[ibig.at[1, g]], ssems[b]).wait()

    for b in range(NBUF):
        start_gather(b, b)

    @pl.loop(0, N_CHUNKS - NBUF, step=NBUF)
    def _chunks(i):
        for b in range(NBUF):
            start_scatter(i + b, b)
        for b in range(NBUF):
            wait_scatter(i + b, b)
            start_gather(i + b + NBUF, b)

    tail = N_CHUNKS - NBUF
    for b in range(NBUF):
        start_scatter(tail + b, b)
    for b in range(NBUF):
        wait_scatter(tail + b, b)

    plsc.subcore_barrier()
    pltpu.sync_copy(acc.at[pl.ds(rbase, ROWS_PER_TILE)],
                    out.at[c, pl.ds(rbase, ROWS_PER_TILE)])


def _make_sc_agg(width):
    mesh = plsc.VectorSubcoreMesh(core_axis_name="c", subcore_axis_name="s")
    return pl.kernel(
        functools.partial(_sc_agg_body, width),
        out_type=jax.ShapeDtypeStruct((NC, N_PAD, width), jnp.float32),
        mesh=mesh,
        scratch_types=[
            pltpu.VMEM_SHARED((N_PAD, width), jnp.float32),
            pltpu.VMEM((2, N_CHUNKS, CHUNK), jnp.int32),
            [pltpu.VMEM((CHUNK, width), jnp.float32) for _ in range(NBUF)],
            [pltpu.SemaphoreType.DMA for _ in range(NBUF)],
            [pltpu.SemaphoreType.DMA for _ in range(NBUF)],
        ],
        compiler_params=pltpu.CompilerParams(use_tc_tiling_on_sc=False),
    )


_sc_agg_80 = _make_sc_agg(W1TAB)
_sc_agg_64 = _make_sc_agg(DH)


def kernel(x, edge_index, W1, b1, W2, b2, Wc, bc):
    edges = edge_index.astype(jnp.int32).reshape(2, NW, N_CHUNKS, CHUNK)
    zrows = jnp.zeros((ROWS_PER_TILE, W1TAB), jnp.float32)
    zrows64 = jnp.zeros((ROWS_PER_TILE, DH), jnp.float32)

    t1 = pl.pallas_call(
        _tc1_body,
        out_shape=jax.ShapeDtypeStruct((N_NODES_C, W1TAB), jnp.float32),
    )(x, W1)

    agg1 = _sc_agg_80(t1, edges, zrows)

    t2 = pl.pallas_call(
        _tc2_body,
        out_shape=jax.ShapeDtypeStruct((N_NODES_C, DH), jnp.float32),
    )(agg1, W2, b1.reshape(1, DH))

    agg2 = _sc_agg_64(t2, edges, zrows64)

    out = pl.pallas_call(
        _tc3_body,
        out_shape=jax.ShapeDtypeStruct((1, 1), jnp.float32),
    )(agg2, agg1, b2.reshape(1, DH), Wc.reshape(1, DH), bc.reshape(1, 1))
    return out


# speedup vs baseline: 18.0229x; 1.0436x over previous
import functools

import jax
import jax.numpy as jnp
from jax import lax
from jax.experimental import pallas as pl
from jax.experimental.pallas import tpu as pltpu
from jax.experimental.pallas import tpu_sc as plsc

N_NODES_C = 10000
N_PAD = 10240
N_EDGES_C = 320000
D1 = 128
DH = 64
W1TAB = 80

NC = 2
NS = 16
NW = NC * NS
EDGES_PER_TILE = N_EDGES_C // NW
CHUNK = 125
N_CHUNKS = EDGES_PER_TILE // CHUNK
ROWS_PER_TILE = N_PAD // NS


def _tc1_body(x_ref, w1_ref, out_ref):
    mm = jnp.dot(x_ref[...], w1_ref[...], preferred_element_type=jnp.float32)
    n = mm.shape[0]
    col = lax.broadcasted_iota(jnp.int32, (n, W1TAB - DH), 1)
    extra = jnp.where(col == 0, 1.0, 0.0).astype(jnp.float32)
    out_ref[...] = jnp.concatenate([mm, extra], axis=1)


def _tc2_body(agg_ref, w2_ref, b1_ref, t2_ref):
    agg = (agg_ref[0] + agg_ref[1])[:N_NODES_C]
    deg = jnp.maximum(agg[:, DH:DH + 1], 1.0)
    h1 = jax.nn.relu(agg[:, :DH] / deg + b1_ref[...])
    t2_ref[...] = jnp.dot(h1, w2_ref[...], preferred_element_type=jnp.float32)


def _tc3_body(agg2_ref, agg1_ref, b2_ref, wc_ref, bc_ref, out_ref):
    agg = (agg2_ref[0] + agg2_ref[1])[:N_NODES_C]
    dcol = (agg1_ref[0] + agg1_ref[1])[:N_NODES_C]
    deg = jnp.maximum(dcol[:, DH:DH + 1], 1.0)
    h2 = jax.nn.relu(agg / deg + b2_ref[...])
    hg = jnp.sum(h2, axis=0, keepdims=True) / N_NODES_C
    out_ref[...] = jnp.sum(hg * wc_ref[...], axis=1, keepdims=True) + bc_ref[...]


def _nbuf(width):
    return 8 if width <= 64 else 5


def _sc_agg_body(width, table, edges, out, acc, ibig, zbuf, rbufs, gsems, ssems):
    NBUF = _nbuf(width)
    c = lax.axis_index("c")
    s = lax.axis_index("s")
    wid = c * NS + s
    rbase = s * ROWS_PER_TILE

    pltpu.sync_copy(edges.at[0, wid], ibig.at[0])
    pltpu.sync_copy(edges.at[1, wid], ibig.at[1])
    @pl.loop(0, 64)
    def _z(i):
        for j in range(width // 16):
            zbuf[i, pl.ds(j * 16, 16)] = jnp.zeros((16,), jnp.float32)
    for k in range(ROWS_PER_TILE // 64):
        pltpu.sync_copy(zbuf, acc.at[pl.ds(rbase + k * 64, 64)])
    plsc.subcore_barrier()

    def start_gather(g, b):
        pltpu.async_copy(table.at[ibig.at[0, g]], rbufs[b], gsems[b])

    def start_scatter(g, b):
        pltpu.make_async_copy(table.at[ibig.at[0, g]], rbufs[b], gsems[b]).wait()
        pltpu.async_copy(rbufs[b], acc.at[ibig.at[1, g]], ssems[b], add=True)

    def wait_scatter(g, b):
        pltpu.make_async_copy(rbufs[b], acc.at[ibig.at[1, g]], ssems[b]).wait()


    for b in range(NBUF):
        start_gather(b, b)

    @pl.loop(0, N_CHUNKS - NBUF, step=NBUF)
    def _chunks(i):
        for b in range(NBUF):
            start_scatter(i + b, b)
        for b in range(NBUF):
            wait_scatter(i + b, b)
            start_gather(i + b + NBUF, b)

    tail = N_CHUNKS - NBUF
    for b in range(NBUF):
        start_scatter(tail + b, b)
    for b in range(NBUF):
        wait_scatter(tail + b, b)

    plsc.subcore_barrier()
    pltpu.sync_copy(acc.at[pl.ds(rbase, ROWS_PER_TILE)],
                    out.at[c, pl.ds(rbase, ROWS_PER_TILE)])


def _make_sc_agg(width):
    mesh = plsc.VectorSubcoreMesh(core_axis_name="c", subcore_axis_name="s")
    nbuf = _nbuf(width)
    return pl.kernel(
        functools.partial(_sc_agg_body, width),
        out_type=jax.ShapeDtypeStruct((NC, N_PAD, width), jnp.float32),
        mesh=mesh,
        scratch_types=[
            pltpu.VMEM_SHARED((N_PAD, width), jnp.float32),
            pltpu.VMEM((2, N_CHUNKS, CHUNK), jnp.int32),
            pltpu.VMEM((64, width), jnp.float32),
            [pltpu.VMEM((CHUNK, width), jnp.float32) for _ in range(nbuf)],
            [pltpu.SemaphoreType.DMA for _ in range(nbuf)],
            [pltpu.SemaphoreType.DMA for _ in range(nbuf)],
        ],
        compiler_params=pltpu.CompilerParams(use_tc_tiling_on_sc=False),
    )


_sc_agg_80 = _make_sc_agg(W1TAB)
_sc_agg_64 = _make_sc_agg(DH)


def kernel(x, edge_index, W1, b1, W2, b2, Wc, bc):
    edges = edge_index.astype(jnp.int32).reshape(2, NW, N_CHUNKS, CHUNK)

    t1 = pl.pallas_call(
        _tc1_body,
        out_shape=jax.ShapeDtypeStruct((N_NODES_C, W1TAB), jnp.float32),
    )(x, W1)

    agg1 = _sc_agg_80(t1, edges)

    t2 = pl.pallas_call(
        _tc2_body,
        out_shape=jax.ShapeDtypeStruct((N_NODES_C, DH), jnp.float32),
    )(agg1, W2, b1.reshape(1, DH))

    agg2 = _sc_agg_64(t2, edges)

    out = pl.pallas_call(
        _tc3_body,
        out_shape=jax.ShapeDtypeStruct((1, 1), jnp.float32),
    )(agg2, agg1, b2.reshape(1, DH), Wc.reshape(1, DH), bc.reshape(1, 1))
    return out


# speedup vs baseline: 21.7581x; 1.2072x over previous
import functools

import jax
import jax.numpy as jnp
from jax import lax
from jax.experimental import pallas as pl
from jax.experimental.pallas import tpu as pltpu
from jax.experimental.pallas import tpu_sc as plsc

N_NODES_C = 10000
N_PAD = 10240
N_EDGES_C = 320000
D1 = 128
DH = 64
W1TAB = 96

NC = 2
NS = 16
NW = NC * NS
EDGES_PER_TILE = N_EDGES_C // NW
CHUNK = 125
N_CHUNKS = EDGES_PER_TILE // CHUNK
ROWS_PER_TILE = N_PAD // NS


def _tc1_body(x_ref, w1_ref, out_ref):
    mm = jnp.dot(x_ref[...], w1_ref[...], preferred_element_type=jnp.float32)
    n = mm.shape[0]
    col = lax.broadcasted_iota(jnp.int32, (n, W1TAB - DH), 1)
    extra = jnp.where(col == 0, 1.0, 0.0).astype(jnp.float32)
    out_ref[...] = jnp.concatenate([mm, extra], axis=1).astype(jnp.bfloat16)


def _tc2_body(agg_ref, w2_ref, b1_ref, t2_ref):
    agg = (agg_ref[0].astype(jnp.float32)
           + agg_ref[1].astype(jnp.float32))[:N_NODES_C]
    deg = jnp.maximum(agg[:, DH:DH + 1], 1.0)
    h1 = jax.nn.relu(agg[:, :DH] / deg + b1_ref[...])
    t2_ref[...] = jnp.dot(
        h1, w2_ref[...], preferred_element_type=jnp.float32).astype(jnp.bfloat16)


def _tc3_body(agg2_ref, agg1_ref, b2_ref, wc_ref, bc_ref, out_ref):
    agg = (agg2_ref[0].astype(jnp.float32)
           + agg2_ref[1].astype(jnp.float32))[:N_NODES_C]
    dcol = (agg1_ref[0].astype(jnp.float32)
            + agg1_ref[1].astype(jnp.float32))[:N_NODES_C]
    deg = jnp.maximum(dcol[:, DH:DH + 1], 1.0)
    h2 = jax.nn.relu(agg / deg + b2_ref[...])
    hg = jnp.sum(h2, axis=0, keepdims=True) / N_NODES_C
    out_ref[...] = jnp.sum(hg * wc_ref[...], axis=1, keepdims=True) + bc_ref[...]


def _nbuf(width):
    return 8


def _sc_agg_body(width, table, edges, out, acc, ibig, zbuf, rbufs, gsems, ssems):
    NBUF = _nbuf(width)
    c = lax.axis_index("c")
    s = lax.axis_index("s")
    wid = c * NS + s
    rbase = s * ROWS_PER_TILE

    pltpu.sync_copy(edges.at[0, wid], ibig.at[0])
    pltpu.sync_copy(edges.at[1, wid], ibig.at[1])
    @pl.loop(0, 64)
    def _z(i):
        for j in range(width // 32):
            zbuf[i, pl.ds(j * 32, 32)] = jnp.zeros((32,), jnp.bfloat16)
    for k in range(ROWS_PER_TILE // 64):
        pltpu.sync_copy(zbuf, acc.at[pl.ds(rbase + k * 64, 64)])
    plsc.subcore_barrier()

    def start_gather(g, b):
        pltpu.async_copy(table.at[ibig.at[0, g]], rbufs[b], gsems[b])

    def start_scatter(g, b):
        pltpu.make_async_copy(table.at[ibig.at[0, g]], rbufs[b], gsems[b]).wait()
        pltpu.async_copy(rbufs[b], acc.at[ibig.at[1, g]], ssems[b], add=True)

    def wait_scatter(g, b):
        pltpu.make_async_copy(rbufs[b], acc.at[ibig.at[1, g]], ssems[b]).wait()


    for b in range(NBUF):
        start_gather(b, b)

    @pl.loop(0, N_CHUNKS - NBUF, step=NBUF)
    def _chunks(i):
        for b in range(NBUF):
            start_scatter(i + b, b)
        for b in range(NBUF):
            wait_scatter(i + b, b)
            start_gather(i + b + NBUF, b)

    tail = N_CHUNKS - NBUF
    for b in range(NBUF):
        start_scatter(tail + b, b)
    for b in range(NBUF):
        wait_scatter(tail + b, b)

    plsc.subcore_barrier()
    pltpu.sync_copy(acc.at[pl.ds(rbase, ROWS_PER_TILE)],
                    out.at[c, pl.ds(rbase, ROWS_PER_TILE)])


def _make_sc_agg(width):
    mesh = plsc.VectorSubcoreMesh(core_axis_name="c", subcore_axis_name="s")
    nbuf = _nbuf(width)
    return pl.kernel(
        functools.partial(_sc_agg_body, width),
        out_type=jax.ShapeDtypeStruct((NC, N_PAD, width), jnp.bfloat16),
        mesh=mesh,
        scratch_types=[
            pltpu.VMEM_SHARED((N_PAD, width), jnp.bfloat16),
            pltpu.VMEM((2, N_CHUNKS, CHUNK), jnp.int32),
            pltpu.VMEM((64, width), jnp.bfloat16),
            [pltpu.VMEM((CHUNK, width), jnp.bfloat16) for _ in range(nbuf)],
            [pltpu.SemaphoreType.DMA for _ in range(nbuf)],
            [pltpu.SemaphoreType.DMA for _ in range(nbuf)],
        ],
        compiler_params=pltpu.CompilerParams(use_tc_tiling_on_sc=False),
    )


_sc_agg_80 = _make_sc_agg(W1TAB)
_sc_agg_64 = _make_sc_agg(DH)


def kernel(x, edge_index, W1, b1, W2, b2, Wc, bc):
    edges = edge_index.astype(jnp.int32).reshape(2, NW, N_CHUNKS, CHUNK)

    t1 = pl.pallas_call(
        _tc1_body,
        out_shape=jax.ShapeDtypeStruct((N_NODES_C, W1TAB), jnp.bfloat16),
    )(x, W1)

    agg1 = _sc_agg_80(t1, edges)

    t2 = pl.pallas_call(
        _tc2_body,
        out_shape=jax.ShapeDtypeStruct((N_NODES_C, DH), jnp.bfloat16),
    )(agg1, W2, b1.reshape(1, DH))

    agg2 = _sc_agg_64(t2, edges)

    out = pl.pallas_call(
        _tc3_body,
        out_shape=jax.ShapeDtypeStruct((1, 1), jnp.float32),
    )(agg2, agg1, b2.reshape(1, DH), Wc.reshape(1, DH), bc.reshape(1, 1))
    return out
